# baseline (speedup 1.0000x reference)
import functools

import jax
import jax.numpy as jnp
from jax import lax
from jax.experimental import pallas as pl
from jax.experimental.pallas import tpu as pltpu
from jax.experimental.pallas import tpu_sc as plsc

VOCAB = 1000000
COL = 64
NC = 2
NS = 16
NW = NC * NS
VCHUNK = 128
NCHUNK = -(-VOCAB // VCHUNK)
PAIRS = NCHUNK * VCHUNK // 2

_mesh = lambda: plsc.VectorSubcoreMesh(core_axis_name="c", subcore_axis_name="s")


def _iota16():
    return lax.iota(jnp.int32, 16)


def _transpose_table():
    @functools.partial(
        pl.kernel,
        mesh=_mesh(),
        out_type=jax.ShapeDtypeStruct((PAIRS, 128), jnp.float32),
        scratch_types=[
            pltpu.VMEM((COL, VCHUNK), jnp.float32),
            pltpu.VMEM((VCHUNK // 2, 128), jnp.float32),
        ],
        compiler_params=pltpu.CompilerParams(needs_layout_passes=False),
    )
    def k(wt_hbm, t2_hbm, stage_v, out_v):
        wid = lax.axis_index("s") * NC + lax.axis_index("c")
        nw = jnp.where(wid < NCHUNK % NW, NCHUNK // NW + 1, NCHUNK // NW)

        def body(i, carry):
            c = wid + i * NW
            v0 = pl.multiple_of(c * VCHUNK, VCHUNK)
            pltpu.sync_copy(wt_hbm.at[:, pl.ds(v0, VCHUNK)], stage_v)
            for p in range(VCHUNK // 2):
                for g in range(8):
                    rows = _iota16() + (16 * g) % 64
                    colv = jnp.full((16,), 2 * p + g // 4, jnp.int32)
                    out_v[p, pl.ds(16 * g, 16)] = plsc.load_gather(
                        stage_v, [rows, colv])
            p0 = pl.multiple_of(c * (VCHUNK // 2), VCHUNK // 2)
            pltpu.sync_copy(out_v, t2_hbm.at[pl.ds(p0, VCHUNK // 2)])
            return carry

        lax.fori_loop(0, nw, body, 0)

    return k


def _gather_out(seq: int, batch: int):
    bw = batch // NW

    @functools.partial(
        pl.kernel,
        mesh=_mesh(),
        out_type=jax.ShapeDtypeStruct((seq, COL, batch), jnp.float32),
        scratch_types=[
            pltpu.VMEM((seq, bw), jnp.int32),
            pltpu.VMEM((bw,), jnp.int32),
            pltpu.VMEM((bw, 128), jnp.float32),
            pltpu.VMEM((COL, bw), jnp.float32),
            pltpu.SemaphoreType.DMA,
        ],
        compiler_params=pltpu.CompilerParams(needs_layout_passes=False),
    )
    def k(xt_hbm, t2_hbm, o_hbm, xv, pidx_v, rows_v, out_v, sem):
        wid = lax.axis_index("s") * NC + lax.axis_index("c")
        b0 = pl.multiple_of(wid * bw, bw)
        pltpu.sync_copy(xt_hbm.at[:, pl.ds(b0, bw)], xv)

        def body(s, carry):
            par64 = []
            for g in range(bw // 16):
                vv = xv[s, pl.ds(16 * g, 16)]
                pidx_v[pl.ds(16 * g, 16)] = lax.shift_right_logical(vv, 1)
                par64.append(lax.shift_left(jnp.bitwise_and(vv, 1), 6))
            pltpu.async_copy(t2_hbm.at[pidx_v], rows_v, sem).wait()
            for c in range(COL):
                for g in range(bw // 16):
                    rows = _iota16() + 16 * g
                    out_v[c, pl.ds(16 * g, 16)] = plsc.load_gather(
                        rows_v, [rows, par64[g] + c])
            pltpu.sync_copy(out_v, o_hbm.at[s, :, pl.ds(b0, bw)])
            return carry

        lax.fori_loop(0, seq, body, 0)

    return k


def kernel(x, embed_weight):
    batch, seq = x.shape
    xt = x.astype(jnp.int32).T
    wt = embed_weight.T
    t2 = _transpose_table()(wt)
    o = _gather_out(seq, batch)(xt, t2)
    return o.transpose(2, 0, 1)


# speedup vs baseline: 2.3598x; 2.3598x over previous
import functools

import jax
import jax.numpy as jnp
from jax import lax
from jax.experimental import pallas as pl
from jax.experimental.pallas import tpu as pltpu
from jax.experimental.pallas import tpu_sc as plsc

VOCAB = 1000000
COL = 64
NC = 2
NS = 16
NW = NC * NS
VCHUNK = 128
NCHUNK = -(-VOCAB // VCHUNK)
PAIRS = NCHUNK * VCHUNK // 2

_mesh = lambda: plsc.VectorSubcoreMesh(core_axis_name="c", subcore_axis_name="s")


def _iota16():
    return lax.iota(jnp.int32, 16)


def _transpose_table():
    @functools.partial(
        pl.kernel,
        mesh=_mesh(),
        out_type=jax.ShapeDtypeStruct((PAIRS, 128), jnp.float32),
        scratch_types=[
            pltpu.VMEM((COL, VCHUNK), jnp.float32),
            pltpu.VMEM((COL, VCHUNK), jnp.float32),
            pltpu.VMEM((VCHUNK // 2, 128), jnp.float32),
            pltpu.VMEM((VCHUNK // 2, 128), jnp.float32),
            pltpu.SemaphoreType.DMA,
            pltpu.SemaphoreType.DMA,
            pltpu.SemaphoreType.DMA,
            pltpu.SemaphoreType.DMA,
        ],
        compiler_params=pltpu.CompilerParams(needs_layout_passes=False),
    )
    def k(wt_hbm, t2_hbm, st0, st1, ov0, ov1, rs0, rs1, ws0, ws1):
        wid = lax.axis_index("s") * NC + lax.axis_index("c")
        nw = jnp.where(wid < NCHUNK % NW, NCHUNK // NW + 1, NCHUNK // NW)

        def chunk_v0(k_ord):
            return pl.multiple_of((wid + k_ord * NW) * VCHUNK, VCHUNK)

        def start_read(k_ord, stage, rsem):
            pltpu.async_copy(
                wt_hbm.at[:, pl.ds(chunk_v0(k_ord), VCHUNK)], stage, rsem)

        def transpose(stage, out):
            rows = [_iota16() + (16 * g) % 64 for g in range(8)]

            @plsc.parallel_loop(0, VCHUNK // 2, unroll=4)
            def _(p):
                for g in range(8):
                    colv = jnp.broadcast_to(2 * p + g // 4, (16,)).astype(
                        jnp.int32)
                    out[p, pl.ds(16 * g, 16)] = plsc.load_gather(
                        stage, [rows[g], colv])

        def slot(k_ord, stage, out, rsem, wsem):
            @pl.when(k_ord < nw)
            def _():
                pltpu.make_async_copy(
                    wt_hbm.at[:, pl.ds(0, VCHUNK)], stage, rsem).wait()

                @pl.when(k_ord >= 2)
                def _():
                    pltpu.make_async_copy(
                        out, t2_hbm.at[pl.ds(0, VCHUNK // 2)], wsem).wait()

                transpose(stage, out)
                p0 = pl.multiple_of(
                    (wid + k_ord * NW) * (VCHUNK // 2), VCHUNK // 2)
                pltpu.async_copy(
                    out, t2_hbm.at[pl.ds(p0, VCHUNK // 2)], wsem)

                @pl.when(k_ord + 2 < nw)
                def _():
                    start_read(k_ord + 2, stage, rsem)

        start_read(0, st0, rs0)
        start_read(1, st1, rs1)

        def body(j, carry):
            slot(2 * j, st0, ov0, rs0, ws0)
            slot(2 * j + 1, st1, ov1, rs1, ws1)
            return carry

        lax.fori_loop(0, (NCHUNK // NW + 2) // 2, body, 0)
        pltpu.make_async_copy(ov0, t2_hbm.at[pl.ds(0, VCHUNK // 2)], ws0).wait()
        pltpu.make_async_copy(ov1, t2_hbm.at[pl.ds(0, VCHUNK // 2)], ws1).wait()

    return k


def _gather_out(seq: int, batch: int):
    bw = batch // NW
    ng = bw // 16

    @functools.partial(
        pl.kernel,
        mesh=_mesh(),
        out_type=jax.ShapeDtypeStruct((seq, COL, batch), jnp.float32),
        scratch_types=[
            pltpu.VMEM((seq, bw), jnp.int32),
            pltpu.VMEM((bw,), jnp.int32),
            pltpu.VMEM((bw,), jnp.int32),
            pltpu.VMEM((bw, 128), jnp.float32),
            pltpu.VMEM((bw, 128), jnp.float32),
            pltpu.VMEM((COL, bw), jnp.float32),
            pltpu.VMEM((COL, bw), jnp.float32),
            pltpu.SemaphoreType.DMA,
            pltpu.SemaphoreType.DMA,
            pltpu.SemaphoreType.DMA,
            pltpu.SemaphoreType.DMA,
        ],
        compiler_params=pltpu.CompilerParams(needs_layout_passes=False),
    )
    def k(xt_hbm, t2_hbm, o_hbm, xv, pi0, pi1, rv0, rv1, ov0, ov1,
          gs0, gs1, ws0, ws1):
        wid = lax.axis_index("s") * NC + lax.axis_index("c")
        b0 = pl.multiple_of(wid * bw, bw)
        pltpu.sync_copy(xt_hbm.at[:, pl.ds(b0, bw)], xv)

        def compute_pidx(s, pidx):
            for g in range(ng):
                vv = xv[s, pl.ds(16 * g, 16)]
                pidx[pl.ds(16 * g, 16)] = lax.shift_right_logical(vv, 1)

        def transpose(s, rows_ref, out):
            rowbase = [_iota16() + 16 * g for g in range(ng)]
            par64 = [
                lax.shift_left(jnp.bitwise_and(xv[s, pl.ds(16 * g, 16)], 1), 6)
                for g in range(ng)
            ]

            @plsc.parallel_loop(0, COL, unroll=4)
            def _(c):
                for g in range(ng):
                    out[c, pl.ds(16 * g, 16)] = plsc.load_gather(
                        rows_ref, [rowbase[g], par64[g] + c])

        def slot(s, pidx, rows_ref, out, gsem, wsem):
            pltpu.make_async_copy(t2_hbm.at[pidx], rows_ref, gsem).wait()

            @pl.when(s >= 2)
            def _():
                pltpu.make_async_copy(
                    out, o_hbm.at[0, :, pl.ds(b0, bw)], wsem).wait()

            transpose(s, rows_ref, out)
            pltpu.async_copy(out, o_hbm.at[s, :, pl.ds(b0, bw)], wsem)

            @pl.when(s + 2 < seq)
            def _():
                compute_pidx(s + 2, pidx)
                pltpu.async_copy(t2_hbm.at[pidx], rows_ref, gsem)

        compute_pidx(0, pi0)
        pltpu.async_copy(t2_hbm.at[pi0], rv0, gs0)
        compute_pidx(1, pi1)
        pltpu.async_copy(t2_hbm.at[pi1], rv1, gs1)

        def body(i, carry):
            slot(2 * i, pi0, rv0, ov0, gs0, ws0)
            slot(2 * i + 1, pi1, rv1, ov1, gs1, ws1)
            return carry

        lax.fori_loop(0, seq // 2, body, 0)
        pltpu.make_async_copy(ov0, o_hbm.at[0, :, pl.ds(b0, bw)], ws0).wait()
        pltpu.make_async_copy(ov1, o_hbm.at[0, :, pl.ds(b0, bw)], ws1).wait()

    return k


def kernel(x, embed_weight):
    batch, seq = x.shape
    xt = x.astype(jnp.int32).T
    wt = embed_weight.T
    t2 = _transpose_table()(wt)
    o = _gather_out(seq, batch)(xt, t2)
    return o.transpose(2, 0, 1)


# speedup vs baseline: 2.3707x; 1.0046x over previous
import functools

import jax
import jax.numpy as jnp
from jax import lax
from jax.experimental import pallas as pl
from jax.experimental.pallas import tpu as pltpu
from jax.experimental.pallas import tpu_sc as plsc

VOCAB = 1000000
COL = 64
NC = 2
NS = 16
NW = NC * NS
VCHUNK = 256
NFULL = VOCAB // VCHUNK
TAILV0 = NFULL * VCHUNK
TAILW = 128
PAIRS = (TAILV0 + TAILW) // 2
VROWS = 2 * PAIRS

_mesh = lambda: plsc.VectorSubcoreMesh(core_axis_name="c", subcore_axis_name="s")


def _iota16():
    return lax.iota(jnp.int32, 16)


def _transpose_table():
    @functools.partial(
        pl.kernel,
        mesh=_mesh(),
        out_type=jax.ShapeDtypeStruct((PAIRS, 128), jnp.float32),
        scratch_types=[
            pltpu.VMEM((COL, VCHUNK), jnp.float32),
            pltpu.VMEM((COL, VCHUNK), jnp.float32),
            pltpu.VMEM((VCHUNK // 2, 128), jnp.float32),
            pltpu.VMEM((VCHUNK // 2, 128), jnp.float32),
            pltpu.VMEM((COL, TAILW), jnp.float32),
            pltpu.VMEM((TAILW // 2, 128), jnp.float32),
            pltpu.SemaphoreType.DMA,
            pltpu.SemaphoreType.DMA,
            pltpu.SemaphoreType.DMA,
            pltpu.SemaphoreType.DMA,
        ],
        compiler_params=pltpu.CompilerParams(needs_layout_passes=False),
    )
    def k(wt_hbm, t2_hbm, st0, st1, ov0, ov1, stt, ovt, rs0, rs1, ws0, ws1):
        wid = lax.axis_index("s") * NC + lax.axis_index("c")
        nw = jnp.where(wid < NFULL % NW, NFULL // NW + 1, NFULL // NW)

        def start_read(k_ord, stage, rsem):
            v0 = pl.multiple_of((wid + k_ord * NW) * VCHUNK, VCHUNK)
            pltpu.async_copy(wt_hbm.at[:, pl.ds(v0, VCHUNK)], stage, rsem)

        def transpose(stage, out, np_):
            rows = [_iota16() + (16 * g) % 64 for g in range(8)]

            @plsc.parallel_loop(0, np_, unroll=4)
            def _(p):
                for g in range(8):
                    colv = jnp.broadcast_to(2 * p + g // 4, (16,)).astype(
                        jnp.int32)
                    out[p, pl.ds(16 * g, 16)] = plsc.load_gather(
                        stage, [rows[g], colv])

        def slot(k_ord, stage, out, rsem, wsem):
            @pl.when(k_ord < nw)
            def _():
                pltpu.make_async_copy(
                    wt_hbm.at[:, pl.ds(0, VCHUNK)], stage, rsem).wait()

                @pl.when(k_ord >= 2)
                def _():
                    pltpu.make_async_copy(
                        out, t2_hbm.at[pl.ds(0, VCHUNK // 2)], wsem).wait()

                transpose(stage, out, VCHUNK // 2)
                p0 = pl.multiple_of(
                    (wid + k_ord * NW) * (VCHUNK // 2), VCHUNK // 2)
                pltpu.async_copy(
                    out, t2_hbm.at[pl.ds(p0, VCHUNK // 2)], wsem)

                @pl.when(k_ord + 2 < nw)
                def _():
                    start_read(k_ord + 2, stage, rsem)

        start_read(0, st0, rs0)
        start_read(1, st1, rs1)

        def body(j, carry):
            slot(2 * j, st0, ov0, rs0, ws0)
            slot(2 * j + 1, st1, ov1, rs1, ws1)
            return carry

        lax.fori_loop(0, (NFULL // NW + 2) // 2, body, 0)

        @pl.when(wid == NW - 1)
        def _():
            t0 = pl.multiple_of(jnp.int32(TAILV0), TAILW)
            pltpu.sync_copy(wt_hbm.at[:, pl.ds(t0, TAILW)], stt)
            transpose(stt, ovt, TAILW // 2)
            pltpu.sync_copy(ovt, t2_hbm.at[pl.ds(TAILV0 // 2, TAILW // 2)])

        pltpu.make_async_copy(ov0, t2_hbm.at[pl.ds(0, VCHUNK // 2)], ws0).wait()
        pltpu.make_async_copy(ov1, t2_hbm.at[pl.ds(0, VCHUNK // 2)], ws1).wait()

    return k


def _gather_out(seq: int, batch: int):
    bw = 128
    nbt = batch // 128
    nst = seq // 8
    ng = bw // 16

    @functools.partial(
        pl.kernel,
        mesh=_mesh(),
        out_type=jax.ShapeDtypeStruct((seq, COL // 8, nbt, 8, 128),
                                      jnp.float32),
        scratch_types=[
            pltpu.VMEM((nst, 8, bw), jnp.int32),
            pltpu.VMEM((bw, COL), jnp.float32),
            pltpu.VMEM((bw, COL), jnp.float32),
            pltpu.VMEM((COL // 8, 8, bw), jnp.float32),
            pltpu.VMEM((COL // 8, 8, bw), jnp.float32),
            pltpu.SemaphoreType.DMA,
            pltpu.SemaphoreType.DMA,
            pltpu.SemaphoreType.DMA,
            pltpu.SemaphoreType.DMA,
        ],
        compiler_params=pltpu.CompilerParams(
            use_tc_tiling_on_sc=False, needs_layout_passes=False),
    )
    def k(x5_hbm, t2_hbm, o5_hbm, xv, rv0, rv1, ov0, ov1, gs0, gs1, ws0, ws1):
        wid = lax.axis_index("s") * NC + lax.axis_index("c")
        pltpu.sync_copy(x5_hbm.at[:, wid], xv)

        def idx_row(s):
            return xv.at[lax.div(s, 8), lax.rem(s, 8)]

        def transpose(rows_ref, out):
            rowbase = [_iota16() + 16 * g for g in range(ng)]

            @plsc.parallel_loop(0, COL, unroll=4)
            def _(c):
                colv = jnp.broadcast_to(c, (16,)).astype(jnp.int32)
                for g in range(ng):
                    out[lax.div(c, 8), lax.rem(c, 8),
                        pl.ds(16 * g, 16)] = plsc.load_gather(
                            rows_ref, [rowbase[g], colv])

        def slot(s, rows_ref, out, gsem, wsem):
            pltpu.make_async_copy(t2_hbm.at[idx_row(s)], rows_ref, gsem).wait()

            @pl.when(s >= 2)
            def _():
                pltpu.make_async_copy(
                    out, o5_hbm.at[0, :, wid], wsem).wait()

            transpose(rows_ref, out)
            pltpu.async_copy(out, o5_hbm.at[s, :, wid], wsem)

            @pl.when(s + 2 < seq)
            def _():
                pltpu.async_copy(t2_hbm.at[idx_row(s + 2)], rows_ref, gsem)

        pltpu.async_copy(t2_hbm.at[idx_row(0)], rv0, gs0)
        pltpu.async_copy(t2_hbm.at[idx_row(1)], rv1, gs1)

        def body(i, carry):
            slot(2 * i, rv0, ov0, gs0, ws0)
            slot(2 * i + 1, rv1, ov1, gs1, ws1)
            return carry

        lax.fori_loop(0, seq // 2, body, 0)
        pltpu.make_async_copy(ov0, o5_hbm.at[0, :, wid], ws0).wait()
        pltpu.make_async_copy(ov1, o5_hbm.at[0, :, wid], ws1).wait()

    return k


def kernel(x, embed_weight):
    batch, seq = x.shape
    x5 = (x.astype(jnp.int32).T
          .reshape(seq // 8, 8, batch // 128, 128)
          .transpose(0, 2, 1, 3))
    wt = embed_weight.T
    t2p = _transpose_table()(wt)
    t2 = t2p.reshape(VROWS, COL)
    o5 = _gather_out(seq, batch)(x5, t2)
    return (o5.transpose(2, 4, 0, 1, 3)
            .reshape(batch, seq, COL))


# speedup vs baseline: 3.4959x; 1.4746x over previous
import functools

import jax
import jax.numpy as jnp
from jax import lax
from jax.experimental import pallas as pl
from jax.experimental.pallas import tpu as pltpu
from jax.experimental.pallas import tpu_sc as plsc

VOCAB = 1000000
COL = 64
NC = 2
NS = 16
NW = NC * NS
VCHUNK = 256
NFULL = VOCAB // VCHUNK
TAILV0 = NFULL * VCHUNK
TAILW = 128
PAIRS = (TAILV0 + TAILW) // 2
VROWS = 2 * PAIRS

_mesh = lambda: plsc.VectorSubcoreMesh(core_axis_name="c", subcore_axis_name="s")


def _iota16():
    return lax.iota(jnp.int32, 16)


def _transpose_table():
    @functools.partial(
        pl.kernel,
        mesh=_mesh(),
        out_type=jax.ShapeDtypeStruct((PAIRS, 128), jnp.float32),
        scratch_types=[
            pltpu.VMEM((COL, VCHUNK + 1), jnp.float32),
            pltpu.VMEM((COL, VCHUNK + 1), jnp.float32),
            pltpu.VMEM((VCHUNK // 2, 128), jnp.float32),
            pltpu.VMEM((VCHUNK // 2, 128), jnp.float32),
            pltpu.VMEM((COL, TAILW + 1), jnp.float32),
            pltpu.VMEM((TAILW // 2, 128), jnp.float32),
            pltpu.SemaphoreType.DMA,
            pltpu.SemaphoreType.DMA,
            pltpu.SemaphoreType.DMA,
            pltpu.SemaphoreType.DMA,
        ],
        compiler_params=pltpu.CompilerParams(needs_layout_passes=False),
    )
    def k(wt_hbm, t2_hbm, st0, st1, ov0, ov1, stt, ovt, rs0, rs1, ws0, ws1):
        wid = lax.axis_index("s") * NC + lax.axis_index("c")
        nw = jnp.where(wid < NFULL % NW, NFULL // NW + 1, NFULL // NW)

        def start_read(k_ord, stage, rsem):
            v0 = pl.multiple_of((wid + k_ord * NW) * VCHUNK, VCHUNK)
            pltpu.async_copy(wt_hbm.at[:, pl.ds(v0, VCHUNK)],
                             stage.at[:, pl.ds(0, VCHUNK)], rsem)

        def transpose(stage, out, np_):
            rows = [_iota16() + (16 * g) % 64 for g in range(8)]

            @plsc.parallel_loop(0, np_, unroll=4)
            def _(p):
                for g in range(8):
                    colv = jnp.broadcast_to(2 * p + g // 4, (16,)).astype(
                        jnp.int32)
                    out[p, pl.ds(16 * g, 16)] = plsc.load_gather(
                        stage, [rows[g], colv])

        def slot(k_ord, stage, out, rsem, wsem):
            @pl.when(k_ord < nw)
            def _():
                pltpu.make_async_copy(
                    wt_hbm.at[:, pl.ds(0, VCHUNK)],
                    stage.at[:, pl.ds(0, VCHUNK)], rsem).wait()

                @pl.when(k_ord >= 2)
                def _():
                    pltpu.make_async_copy(
                        out, t2_hbm.at[pl.ds(0, VCHUNK // 2)], wsem).wait()

                transpose(stage, out, VCHUNK // 2)
                p0 = pl.multiple_of(
                    (wid + k_ord * NW) * (VCHUNK // 2), VCHUNK // 2)
                pltpu.async_copy(
                    out, t2_hbm.at[pl.ds(p0, VCHUNK // 2)], wsem)

                @pl.when(k_ord + 2 < nw)
                def _():
                    start_read(k_ord + 2, stage, rsem)

        start_read(0, st0, rs0)
        start_read(1, st1, rs1)

        def body(j, carry):
            slot(2 * j, st0, ov0, rs0, ws0)
            slot(2 * j + 1, st1, ov1, rs1, ws1)
            return carry

        lax.fori_loop(0, (NFULL // NW + 2) // 2, body, 0)

        @pl.when(wid == NW - 1)
        def _():
            t0 = pl.multiple_of(jnp.int32(TAILV0), TAILW)
            pltpu.sync_copy(wt_hbm.at[:, pl.ds(t0, TAILW)],
                            stt.at[:, pl.ds(0, TAILW)])
            transpose(stt, ovt, TAILW // 2)
            pltpu.sync_copy(ovt, t2_hbm.at[pl.ds(TAILV0 // 2, TAILW // 2)])

        pltpu.make_async_copy(ov0, t2_hbm.at[pl.ds(0, VCHUNK // 2)], ws0).wait()
        pltpu.make_async_copy(ov1, t2_hbm.at[pl.ds(0, VCHUNK // 2)], ws1).wait()

    return k


def _gather_out(seq: int, batch: int):
    bw = 128
    nbt = batch // 128
    nst = seq // 8
    ng = bw // 16

    @functools.partial(
        pl.kernel,
        mesh=_mesh(),
        out_type=jax.ShapeDtypeStruct((seq, COL // 8, nbt, 8, 128),
                                      jnp.float32),
        scratch_types=[
            pltpu.VMEM((nst, 8, bw), jnp.int32),
            pltpu.VMEM((bw, COL), jnp.float32),
            pltpu.VMEM((bw, COL), jnp.float32),
            pltpu.VMEM((COL // 8, 8, bw + 5), jnp.float32),
            pltpu.VMEM((COL // 8, 8, bw + 5), jnp.float32),
            pltpu.SemaphoreType.DMA,
            pltpu.SemaphoreType.DMA,
            pltpu.SemaphoreType.DMA,
            pltpu.SemaphoreType.DMA,
        ],
        compiler_params=pltpu.CompilerParams(
            use_tc_tiling_on_sc=False, needs_layout_passes=False),
    )
    def k(x5_hbm, t2_hbm, o5_hbm, xv, rv0, rv1, ov0, ov1, gs0, gs1, ws0, ws1):
        wid = lax.axis_index("s") * NC + lax.axis_index("c")
        pltpu.sync_copy(x5_hbm.at[:, wid], xv)

        def idx_row(s):
            return xv.at[lax.div(s, 8), lax.rem(s, 8)]

        def transpose(rows_ref, out):
            i0 = [lax.shift_right_logical(_iota16() + 16 * g2, 3)
                  for g2 in range(COL // 16)]
            i1 = [jnp.bitwise_and(_iota16() + 16 * g2, 7)
                  for g2 in range(COL // 16)]

            @plsc.parallel_loop(0, bw, unroll=4)
            def _(b):
                bidx = jnp.broadcast_to(b, (16,)).astype(jnp.int32)
                for g2 in range(COL // 16):
                    plsc.store_scatter(out, [i0[g2], i1[g2], bidx],
                                       rows_ref[b, pl.ds(16 * g2, 16)])

        def slot(s, rows_ref, out, gsem, wsem):
            pltpu.make_async_copy(t2_hbm.at[idx_row(s)], rows_ref, gsem).wait()

            @pl.when(s >= 2)
            def _():
                pltpu.make_async_copy(
                    out.at[:, :, pl.ds(0, bw)], o5_hbm.at[0, :, wid],
                    wsem).wait()

            transpose(rows_ref, out)
            pltpu.async_copy(out.at[:, :, pl.ds(0, bw)],
                             o5_hbm.at[s, :, wid], wsem)

            @pl.when(s + 2 < seq)
            def _():
                pltpu.async_copy(t2_hbm.at[idx_row(s + 2)], rows_ref, gsem)

        pltpu.async_copy(t2_hbm.at[idx_row(0)], rv0, gs0)
        pltpu.async_copy(t2_hbm.at[idx_row(1)], rv1, gs1)

        def body(i, carry):
            slot(2 * i, rv0, ov0, gs0, ws0)
            slot(2 * i + 1, rv1, ov1, gs1, ws1)
            return carry

        lax.fori_loop(0, seq // 2, body, 0)
        pltpu.make_async_copy(
            ov0.at[:, :, pl.ds(0, bw)], o5_hbm.at[0, :, wid], ws0).wait()
        pltpu.make_async_copy(
            ov1.at[:, :, pl.ds(0, bw)], o5_hbm.at[0, :, wid], ws1).wait()

    return k


def kernel(x, embed_weight):
    batch, seq = x.shape
    x5 = (x.astype(jnp.int32).T
          .reshape(seq // 8, 8, batch // 128, 128)
          .transpose(0, 2, 1, 3))
    wt = embed_weight.T
    t2p = _transpose_table()(wt)
    t2 = t2p.reshape(VROWS, COL)
    o5 = _gather_out(seq, batch)(x5, t2)
    return (o5.transpose(2, 4, 0, 1, 3)
            .reshape(batch, seq, COL))
